# Initial kernel scaffold; baseline (speedup 1.0000x reference)
import functools

import jax
import jax.numpy as jnp
from jax import lax
from jax.experimental import pallas as pl
from jax.experimental.pallas import tpu as pltpu
from jax.experimental.pallas import tpu_sc as plsc

B, N, S, K, C = 16, 4096, 512, 32, 64
QC = 128
CH = 256
NCH = N // CH
F32 = jnp.float32
I32 = jnp.int32

NC_SC, NS_SC = 2, 16
NW = NC_SC * NS_SC
ROWS = B * K * S
ROWS_W = ROWS // NW
CHUNK = 128
NCHUNK = ROWS_W // CHUNK

_HI = jax.lax.Precision.HIGHEST


def _fps_body(xt_ref, out_ref, dist_ref, far_ref):
    i = pl.program_id(0)

    @pl.when(i == 0)
    def _init():
        dist_ref[...] = jnp.full((B, N), 1e10, F32)
        far_ref[...] = jnp.zeros((B, 128), I32)

    x = xt_ref[0]
    y = xt_ref[1]
    z = xt_ref[2]
    far = far_ref[:, 0:1]
    lane = lax.broadcasted_iota(I32, (B, N), 1)
    oh = lane == far
    ninf = jnp.float32(-jnp.inf)
    cx = jnp.max(jnp.where(oh, x, ninf), axis=1, keepdims=True)
    cy = jnp.max(jnp.where(oh, y, ninf), axis=1, keepdims=True)
    cz = jnp.max(jnp.where(oh, z, ninf), axis=1, keepdims=True)
    out_ref[...] = jnp.concatenate(
        [cx[None], cy[None], cz[None]], axis=0)
    dx = x - cx
    dy = y - cy
    dz = z - cz
    d = dx * dx + dy * dy + dz * dz
    dist = dist_ref[...]
    dist = jnp.where(d < dist, d, dist)
    dist_ref[...] = dist
    m = jnp.max(dist, axis=1, keepdims=True)
    nxt = jnp.min(jnp.where(dist == m, lane, jnp.int32(N)),
                  axis=1, keepdims=True)
    far_ref[...] = jnp.broadcast_to(nxt, (B, 128))


def _fps(xt):
    return pl.pallas_call(
        _fps_body,
        grid=(S,),
        in_specs=[pl.BlockSpec((3, B, N), lambda i: (0, 0, 0))],
        out_specs=pl.BlockSpec((3, B, 1), lambda i: (0, 0, i)),
        out_shape=jax.ShapeDtypeStruct((3, B, S), F32),
        scratch_shapes=[pltpu.VMEM((B, N), F32), pltpu.VMEM((B, 128), I32)],
    )(xt)


def _g0_body(xyz_ref, pts_ref, w0t_ref, g0_ref):
    xyz = xyz_ref[0]
    pts = pts_ref[0]
    w = w0t_ref[...]
    g = (jnp.dot(xyz, w[0:3], precision=_HI, preferred_element_type=F32)
         + jnp.dot(pts, w[3:], precision=_HI, preferred_element_type=F32))
    g0_ref[...] = g[None]


def _g0(xyz, points, w0t):
    return pl.pallas_call(
        _g0_body,
        grid=(B,),
        in_specs=[
            pl.BlockSpec((1, N, 3), lambda b: (b, 0, 0)),
            pl.BlockSpec((1, N, C), lambda b: (b, 0, 0)),
            pl.BlockSpec((C + 3, C), lambda b: (0, 0)),
        ],
        out_specs=pl.BlockSpec((1, N, C), lambda b: (b, 0, 0)),
        out_shape=jax.ShapeDtypeStruct((B, N, C), F32),
    )(xyz, points, w0t)


def _sel_body(xyz_ref, nxt_ref, idx_ref, scr_ref):
    b = pl.program_id(0)
    xq = nxt_ref[0]
    yq = nxt_ref[1]
    zq = nxt_ref[2]
    for c in range(NCH):
        p = xyz_ref[0, pl.ds(c * CH, CH), :]
        dx = p[:, 0:1] - xq
        dy = p[:, 1:2] - yq
        dz = p[:, 2:3] - zq
        d = dx * dx + dy * dy + dz * dz
        bits = lax.bitcast_convert_type(d, I32)
        sub = lax.broadcasted_iota(I32, (CH, QC), 0) + c * CH
        scr_ref[pl.ds(c * CH, CH), :] = (bits & jnp.int32(-4096)) | sub

    maxi = jnp.int32(0x7FFFFFFF)

    def step(k, prev):
        def chunk(c, m):
            v = scr_ref[pl.ds(c * CH, CH), :]
            v = jnp.where(v > prev, v, maxi)
            return jnp.minimum(m, jnp.min(v, axis=0, keepdims=True))

        m = lax.fori_loop(0, NCH, chunk, jnp.full((1, QC), maxi, I32))
        idx_ref[0, pl.ds(k, 1), :] = (m & 4095) + b * N
        return m

    lax.fori_loop(0, K, step, jnp.full((1, QC), jnp.int32(-1), I32))


def _select(xyz, nxt):
    return pl.pallas_call(
        _sel_body,
        grid=(B, S // QC),
        in_specs=[
            pl.BlockSpec((1, N, 3), lambda b, q: (b, 0, 0)),
            pl.BlockSpec((3, 1, QC), lambda b, q: (0, b, q)),
        ],
        out_specs=pl.BlockSpec((1, K, QC), lambda b, q: (b, 0, q)),
        out_shape=jax.ShapeDtypeStruct((B, K, S), I32),
        scratch_shapes=[pltpu.VMEM((N, QC), I32)],
    )(xyz, nxt)


def _sc_gather(table, idx3):
    mesh = plsc.VectorSubcoreMesh(core_axis_name="c", subcore_axis_name="s")

    @functools.partial(
        pl.kernel,
        mesh=mesh,
        out_type=jax.ShapeDtypeStruct((ROWS, C), F32),
        scratch_types=[
            pltpu.VMEM((NCHUNK, CHUNK), I32),
            pltpu.VMEM((CHUNK, C), F32),
            pltpu.SemaphoreType.DMA,
        ],
    )
    def k(table_hbm, idx_hbm, out_hbm, idx_v, buf, sem):
        wid = lax.axis_index("s") * NC_SC + lax.axis_index("c")
        base = wid * ROWS_W
        pltpu.sync_copy(idx_hbm.at[wid], idx_v)

        def body(c, carry):
            pltpu.async_copy(table_hbm.at[idx_v.at[c]], buf, sem).wait()
            pltpu.sync_copy(buf, out_hbm.at[pl.ds(base + c * CHUNK, CHUNK)])
            return carry

        lax.fori_loop(0, NCHUNK, body, 0)

    return k(table, idx3)


def _mlp_body(g_ref, nxy_ref, w0t_ref, b0_ref, w1t_ref, b1_ref,
              w2t_ref, b2_ref, out_ref):
    nxy = nxy_ref[0]
    q0 = (b0_ref[...][None, :]
          - jnp.dot(nxy, w0t_ref[0:3], precision=_HI,
                    preferred_element_type=F32))
    b1 = b1_ref[...][None, :]
    b2 = b2_ref[...][None, :]
    w1 = w1t_ref[...]
    w2 = w2t_ref[...]
    acc = jnp.full((S, 2 * C), -jnp.inf, F32)
    for k in range(K):
        a = jnp.maximum(g_ref[0, k] + q0, 0.0)
        h = jnp.maximum(
            jnp.dot(a, w1, precision=_HI, preferred_element_type=F32) + b1,
            0.0)
        o = jnp.maximum(
            jnp.dot(h, w2, precision=_HI, preferred_element_type=F32) + b2,
            0.0)
        acc = jnp.maximum(acc, o)
    out_ref[...] = acc[None]


def _mlp(g, new_xyz, w0t, b0, w1t, b1, w2t, b2):
    return pl.pallas_call(
        _mlp_body,
        grid=(B,),
        in_specs=[
            pl.BlockSpec((1, K, S, C), lambda b: (b, 0, 0, 0)),
            pl.BlockSpec((1, S, 3), lambda b: (b, 0, 0)),
            pl.BlockSpec((C + 3, C), lambda b: (0, 0)),
            pl.BlockSpec((C,), lambda b: (0,)),
            pl.BlockSpec((C, C), lambda b: (0, 0)),
            pl.BlockSpec((C,), lambda b: (0,)),
            pl.BlockSpec((C, 2 * C), lambda b: (0, 0)),
            pl.BlockSpec((2 * C,), lambda b: (0,)),
        ],
        out_specs=pl.BlockSpec((1, S, 2 * C), lambda b: (b, 0, 0)),
        out_shape=jax.ShapeDtypeStruct((B, S, 2 * C), F32),
    )(g, new_xyz, w0t, b0, w1t, b1, w2t, b2)


def kernel(xyz, points, W0, b0, W1, b1, W2, b2):
    xt = jnp.transpose(xyz, (2, 0, 1))
    nxt = _fps(xt)
    new_xyz = jnp.transpose(nxt, (1, 2, 0))
    w0t = W0.T
    w1t = W1.T
    w2t = W2.T
    g0 = _g0(xyz, points, w0t)
    idx = _select(xyz, nxt)
    g = _sc_gather(g0.reshape(B * N, C),
                   idx.reshape(NW, NCHUNK, CHUNK))
    out = _mlp(g.reshape(B, K, S, C), new_xyz,
               w0t, b0, w1t, b1, w2t, b2)
    return (new_xyz, out)


# speedup vs baseline: 13.9341x; 13.9341x over previous
import functools

import jax
import jax.numpy as jnp
from jax import lax
from jax.experimental import pallas as pl
from jax.experimental.pallas import tpu as pltpu
from jax.experimental.pallas import tpu_sc as plsc

B, N, S, K, C = 16, 4096, 512, 32, 64
QC = 128
CH = 256
NCH = N // CH
F32 = jnp.float32
I32 = jnp.int32

NC_SC, NS_SC = 2, 16
NW = NC_SC * NS_SC
ROWS = B * K * S
ROWS_W = ROWS // NW
CHUNK = 128
NCHUNK = ROWS_W // CHUNK

_HI = jax.lax.Precision.HIGHEST


def _fps_body(xt_ref, out_ref, dist_ref, far_ref):
    i = pl.program_id(0)

    @pl.when(i == 0)
    def _init():
        dist_ref[...] = jnp.full((B, N), 1e10, F32)
        far_ref[...] = jnp.zeros((B, 128), I32)

    x = xt_ref[0]
    y = xt_ref[1]
    z = xt_ref[2]
    far = far_ref[:, 0:1]
    lane = lax.broadcasted_iota(I32, (B, N), 1)
    oh = lane == far
    ninf = jnp.float32(-jnp.inf)
    cx = jnp.max(jnp.where(oh, x, ninf), axis=1, keepdims=True)
    cy = jnp.max(jnp.where(oh, y, ninf), axis=1, keepdims=True)
    cz = jnp.max(jnp.where(oh, z, ninf), axis=1, keepdims=True)
    out_ref[...] = jnp.concatenate([cx, cy, cz], axis=1)[None]
    dx = x - cx
    dy = y - cy
    dz = z - cz
    d = dx * dx + dy * dy + dz * dz
    dist = dist_ref[...]
    dist = jnp.where(d < dist, d, dist)
    dist_ref[...] = dist
    m = jnp.max(dist, axis=1, keepdims=True)
    nxt = jnp.min(jnp.where(dist == m, lane, jnp.int32(N)),
                  axis=1, keepdims=True)
    far_ref[...] = jnp.broadcast_to(nxt, (B, 128))


def _fps(xt):
    return pl.pallas_call(
        _fps_body,
        grid=(S,),
        in_specs=[pl.BlockSpec((3, B, N), lambda i: (0, 0, 0))],
        out_specs=pl.BlockSpec((1, B, 3), lambda i: (i, 0, 0)),
        out_shape=jax.ShapeDtypeStruct((S, B, 3), F32),
        scratch_shapes=[pltpu.VMEM((B, N), F32), pltpu.VMEM((B, 128), I32)],
    )(xt)


def _g0_body(xyz_ref, pts_ref, w0t_ref, g0_ref):
    xyz = xyz_ref[0]
    pts = pts_ref[0]
    w = w0t_ref[...]
    g = (jnp.dot(xyz, w[0:3], precision=_HI, preferred_element_type=F32)
         + jnp.dot(pts, w[3:], precision=_HI, preferred_element_type=F32))
    g0_ref[...] = g[None]


def _g0(xyz, points, w0t):
    return pl.pallas_call(
        _g0_body,
        grid=(B,),
        in_specs=[
            pl.BlockSpec((1, N, 3), lambda b: (b, 0, 0)),
            pl.BlockSpec((1, N, C), lambda b: (b, 0, 0)),
            pl.BlockSpec((C + 3, C), lambda b: (0, 0)),
        ],
        out_specs=pl.BlockSpec((1, N, C), lambda b: (b, 0, 0)),
        out_shape=jax.ShapeDtypeStruct((B, N, C), F32),
    )(xyz, points, w0t)


def _sel_body(xyz_ref, nxt_ref, idx_ref, scr_ref):
    b = pl.program_id(0)
    q3 = nxt_ref[0]
    xq = q3[0:1, :]
    yq = q3[1:2, :]
    zq = q3[2:3, :]
    qsum = xq * xq + yq * yq + zq * zq
    q3b = q3.astype(jnp.bfloat16)
    for c in range(NCH):
        p = xyz_ref[0, pl.ds(c * CH, CH), :]
        psum = (p[:, 0:1] * p[:, 0:1] + p[:, 1:2] * p[:, 1:2]
                + p[:, 2:3] * p[:, 2:3])
        mm = jnp.dot(p.astype(jnp.bfloat16), q3b,
                     preferred_element_type=F32)
        d = -2.0 * mm
        d = d + qsum
        d = d + psum
        bits = lax.bitcast_convert_type(d, I32)
        sub = lax.broadcasted_iota(I32, (CH, QC), 0) + c * CH
        scr_ref[pl.ds(c * CH, CH), :] = (bits & jnp.int32(-4096)) | sub

    maxi = jnp.int32(0x7FFFFFFF)

    def step(k, prev):
        def chunk(c, m):
            v = scr_ref[pl.ds(c * CH, CH), :]
            v = jnp.where(v > prev, v, maxi)
            return jnp.minimum(m, jnp.min(v, axis=0, keepdims=True))

        m = lax.fori_loop(0, NCH, chunk, jnp.full((1, QC), maxi, I32))
        idx_ref[0, pl.ds(k, 1), :] = (m & 4095) + b * N
        return m

    lax.fori_loop(0, K, step, jnp.full((1, QC), jnp.int32(-(2**31)), I32))


def _select(xyz, nxt):
    return pl.pallas_call(
        _sel_body,
        grid=(B, S // QC),
        in_specs=[
            pl.BlockSpec((1, N, 3), lambda b, q: (b, 0, 0)),
            pl.BlockSpec((1, 3, QC), lambda b, q: (b, 0, q)),
        ],
        out_specs=pl.BlockSpec((1, K, QC), lambda b, q: (b, 0, q)),
        out_shape=jax.ShapeDtypeStruct((B, K, S), I32),
        scratch_shapes=[pltpu.VMEM((N, QC), I32)],
    )(xyz, nxt)


def _sc_gather(table, idx3):
    mesh = plsc.VectorSubcoreMesh(core_axis_name="c", subcore_axis_name="s")

    @functools.partial(
        pl.kernel,
        mesh=mesh,
        compiler_params=pltpu.CompilerParams(use_tc_tiling_on_sc=False),
        out_type=jax.ShapeDtypeStruct((ROWS, C), F32),
        scratch_types=[
            pltpu.VMEM((NCHUNK, CHUNK), I32),
            pltpu.VMEM((CHUNK, C), F32),
            pltpu.SemaphoreType.DMA,
        ],
    )
    def k(table_hbm, idx_hbm, out_hbm, idx_v, buf, sem):
        wid = lax.axis_index("s") * NC_SC + lax.axis_index("c")
        base = wid * ROWS_W
        pltpu.sync_copy(idx_hbm.at[wid], idx_v)

        def body(c, carry):
            pltpu.async_copy(table_hbm.at[idx_v.at[c]], buf, sem).wait()
            pltpu.sync_copy(buf, out_hbm.at[pl.ds(base + c * CHUNK, CHUNK)])
            return carry

        lax.fori_loop(0, NCHUNK, body, 0)

    return k(table, idx3)


def _mlp_body(g_ref, nxy_ref, w0t_ref, b0_ref, w1t_ref, b1_ref,
              w2t_ref, b2_ref, out_ref):
    nxy = nxy_ref[0]
    q0 = (b0_ref[...][None, :]
          - jnp.dot(nxy, w0t_ref[0:3], precision=_HI,
                    preferred_element_type=F32))
    b1 = b1_ref[...][None, :]
    b2 = b2_ref[...][None, :]
    w1 = w1t_ref[...]
    w2 = w2t_ref[...]
    acc = jnp.full((S, 2 * C), -jnp.inf, F32)
    for k in range(K):
        a = jnp.maximum(g_ref[0, k] + q0, 0.0)
        h = jnp.maximum(
            jnp.dot(a, w1, precision=_HI, preferred_element_type=F32) + b1,
            0.0)
        o = jnp.maximum(
            jnp.dot(h, w2, precision=_HI, preferred_element_type=F32) + b2,
            0.0)
        acc = jnp.maximum(acc, o)
    out_ref[...] = acc[None]


def _mlp(g, new_xyz, w0t, b0, w1t, b1, w2t, b2):
    return pl.pallas_call(
        _mlp_body,
        grid=(B,),
        in_specs=[
            pl.BlockSpec((1, K, S, C), lambda b: (b, 0, 0, 0)),
            pl.BlockSpec((1, S, 3), lambda b: (b, 0, 0)),
            pl.BlockSpec((C + 3, C), lambda b: (0, 0)),
            pl.BlockSpec((C,), lambda b: (0,)),
            pl.BlockSpec((C, C), lambda b: (0, 0)),
            pl.BlockSpec((C,), lambda b: (0,)),
            pl.BlockSpec((C, 2 * C), lambda b: (0, 0)),
            pl.BlockSpec((2 * C,), lambda b: (0,)),
        ],
        out_specs=pl.BlockSpec((1, S, 2 * C), lambda b: (b, 0, 0)),
        out_shape=jax.ShapeDtypeStruct((B, S, 2 * C), F32),
    )(g, new_xyz, w0t, b0, w1t, b1, w2t, b2)


def kernel(xyz, points, W0, b0, W1, b1, W2, b2):
    xt = jnp.transpose(xyz, (2, 0, 1))
    nxt = _fps(xt)
    new_xyz = jnp.transpose(nxt, (1, 0, 2))
    w0t = W0.T
    w1t = W1.T
    w2t = W2.T
    g0 = _g0(xyz, points, w0t)
    idx = _select(xyz, jnp.transpose(nxt, (1, 2, 0)))
    g = _sc_gather(g0.reshape(B * N, C),
                   idx.reshape(NW, NCHUNK, CHUNK))
    out = _mlp(g.reshape(B, K, S, C), new_xyz,
               w0t, b0, w1t, b1, w2t, b2)
    return (new_xyz, out)


# speedup vs baseline: 15.4426x; 1.1083x over previous
import functools

import jax
import jax.numpy as jnp
from jax import lax
from jax.experimental import pallas as pl
from jax.experimental.pallas import tpu as pltpu
from jax.experimental.pallas import tpu_sc as plsc

B, N, S, K, C = 16, 4096, 512, 32, 64
QC = 128
CH = 256
NCH = N // CH
F32 = jnp.float32
I32 = jnp.int32

NC_SC, NS_SC = 2, 16
NW = NC_SC * NS_SC
ROWS = B * K * S
ROWS_W = ROWS // NW
CHUNK = 128
NCHUNK = ROWS_W // CHUNK

_HI = jax.lax.Precision.HIGHEST


def _fps_body(xt_ref, out_ref, dist_ref, far_ref):
    i = pl.program_id(0)

    @pl.when(i == 0)
    def _init():
        dist_ref[...] = jnp.full((B, N), 1e10, F32)
        far_ref[...] = jnp.zeros((B, 128), I32)

    x = xt_ref[0]
    y = xt_ref[1]
    z = xt_ref[2]
    far = far_ref[:, 0:1]
    lane = lax.broadcasted_iota(I32, (B, N), 1)
    oh = lane == far
    ninf = jnp.float32(-jnp.inf)
    cx = jnp.max(jnp.where(oh, x, ninf), axis=1, keepdims=True)
    cy = jnp.max(jnp.where(oh, y, ninf), axis=1, keepdims=True)
    cz = jnp.max(jnp.where(oh, z, ninf), axis=1, keepdims=True)
    out_ref[...] = jnp.concatenate([cx, cy, cz], axis=1)[None]
    dx = x - cx
    dy = y - cy
    dz = z - cz
    d = dx * dx + dy * dy + dz * dz
    dist = dist_ref[...]
    dist = jnp.where(d < dist, d, dist)
    dist_ref[...] = dist
    m = jnp.max(dist, axis=1, keepdims=True)
    nxt = jnp.min(jnp.where(dist == m, lane, jnp.int32(N)),
                  axis=1, keepdims=True)
    far_ref[...] = jnp.broadcast_to(nxt, (B, 128))


def _fps(xt):
    return pl.pallas_call(
        _fps_body,
        grid=(S,),
        in_specs=[pl.BlockSpec((3, B, N), lambda i: (0, 0, 0))],
        out_specs=pl.BlockSpec((1, B, 3), lambda i: (i, 0, 0)),
        out_shape=jax.ShapeDtypeStruct((S, B, 3), F32),
        scratch_shapes=[pltpu.VMEM((B, N), F32), pltpu.VMEM((B, 128), I32)],
    )(xt)


def _g0_body(xyz_ref, pts_ref, w0t_ref, g0_ref):
    xyz = xyz_ref[0]
    pts = pts_ref[0]
    w = w0t_ref[...]
    g = (jnp.dot(xyz, w[0:3], precision=_HI, preferred_element_type=F32)
         + jnp.dot(pts, w[3:], precision=_HI, preferred_element_type=F32))
    g0_ref[...] = g[None]


def _g0(xyz, points, w0t):
    return pl.pallas_call(
        _g0_body,
        grid=(B,),
        in_specs=[
            pl.BlockSpec((1, N, 3), lambda b: (b, 0, 0)),
            pl.BlockSpec((1, N, C), lambda b: (b, 0, 0)),
            pl.BlockSpec((C + 3, C), lambda b: (0, 0)),
        ],
        out_specs=pl.BlockSpec((1, N, C), lambda b: (b, 0, 0)),
        out_shape=jax.ShapeDtypeStruct((B, N, C), F32),
    )(xyz, points, w0t)


def _sel_body(xyz_ref, nxt_ref, idx_ref, scr_ref):
    b = pl.program_id(0)
    q3 = nxt_ref[0]
    xq = q3[0:1, :]
    yq = q3[1:2, :]
    zq = q3[2:3, :]
    qsum = xq * xq + yq * yq + zq * zq
    q3b = q3.astype(jnp.bfloat16)
    for c in range(NCH):
        p = xyz_ref[0, pl.ds(c * CH, CH), :]
        psum = (p[:, 0:1] * p[:, 0:1] + p[:, 1:2] * p[:, 1:2]
                + p[:, 2:3] * p[:, 2:3])
        mm = jnp.dot(p.astype(jnp.bfloat16), q3b,
                     preferred_element_type=F32)
        d = -2.0 * mm
        d = d + qsum
        d = d + psum
        bits = lax.bitcast_convert_type(d, I32)
        sub = lax.broadcasted_iota(I32, (CH, QC), 0) + c * CH
        scr_ref[pl.ds(c * CH, CH), :] = (bits & jnp.int32(-4096)) | sub

    maxi = jnp.int32(0x7FFFFFFF)

    def step(k, prev):
        parts = []
        for c in range(NCH):
            v = scr_ref[pl.ds(c * CH, CH), :]
            v = jnp.where(v > prev, v, maxi)
            parts.append(jnp.min(v, axis=0, keepdims=True))
        while len(parts) > 1:
            parts = [jnp.minimum(parts[i], parts[i + 1])
                     for i in range(0, len(parts) - 1, 2)] + (
                         [parts[-1]] if len(parts) % 2 else [])
        m = parts[0]
        idx_ref[0, pl.ds(k, 1), :] = (m & 4095) + b * N
        return m

    lax.fori_loop(0, K, step, jnp.full((1, QC), jnp.int32(-(2**31)), I32))


def _select(xyz, nxt):
    return pl.pallas_call(
        _sel_body,
        grid=(B, S // QC),
        in_specs=[
            pl.BlockSpec((1, N, 3), lambda b, q: (b, 0, 0)),
            pl.BlockSpec((1, 3, QC), lambda b, q: (b, 0, q)),
        ],
        out_specs=pl.BlockSpec((1, K, QC), lambda b, q: (b, 0, q)),
        out_shape=jax.ShapeDtypeStruct((B, K, S), I32),
        scratch_shapes=[pltpu.VMEM((N, QC), I32)],
    )(xyz, nxt)


def _sc_gather(table, idx3):
    mesh = plsc.VectorSubcoreMesh(core_axis_name="c", subcore_axis_name="s")

    @functools.partial(
        pl.kernel,
        mesh=mesh,
        compiler_params=pltpu.CompilerParams(use_tc_tiling_on_sc=False),
        out_type=jax.ShapeDtypeStruct((ROWS, C), F32),
        scratch_types=[
            pltpu.VMEM((NCHUNK, CHUNK), I32),
            pltpu.VMEM((CHUNK, C), F32),
            pltpu.SemaphoreType.DMA,
        ],
    )
    def k(table_hbm, idx_hbm, out_hbm, idx_v, buf, sem):
        wid = lax.axis_index("s") * NC_SC + lax.axis_index("c")
        base = wid * ROWS_W
        pltpu.sync_copy(idx_hbm.at[wid], idx_v)

        def body(c, carry):
            pltpu.async_copy(table_hbm.at[idx_v.at[c]], buf, sem).wait()
            pltpu.sync_copy(buf, out_hbm.at[pl.ds(base + c * CHUNK, CHUNK)])
            return carry

        lax.fori_loop(0, NCHUNK, body, 0)

    return k(table, idx3)


def _mlp_body(g_ref, nxy_ref, w0t_ref, b0_ref, w1t_ref, b1_ref,
              w2t_ref, b2_ref, out_ref):
    nxy = nxy_ref[0]
    q0 = (b0_ref[...][None, :]
          - jnp.dot(nxy, w0t_ref[0:3], precision=_HI,
                    preferred_element_type=F32))
    b1 = b1_ref[...][None, :]
    b2 = b2_ref[...][None, :]
    w1 = w1t_ref[...]
    w2 = w2t_ref[...]
    a = jnp.maximum(g_ref[0] + q0[None], 0.0).reshape(K * S, C)
    h = jnp.maximum(
        jnp.dot(a, w1, precision=_HI, preferred_element_type=F32) + b1, 0.0)
    o = jnp.maximum(
        jnp.dot(h, w2, precision=_HI, preferred_element_type=F32) + b2, 0.0)
    out_ref[...] = jnp.max(o.reshape(K, S, 2 * C), axis=0)[None]


def _mlp(g, new_xyz, w0t, b0, w1t, b1, w2t, b2):
    return pl.pallas_call(
        _mlp_body,
        grid=(B,),
        in_specs=[
            pl.BlockSpec((1, K, S, C), lambda b: (b, 0, 0, 0)),
            pl.BlockSpec((1, S, 3), lambda b: (b, 0, 0)),
            pl.BlockSpec((C + 3, C), lambda b: (0, 0)),
            pl.BlockSpec((C,), lambda b: (0,)),
            pl.BlockSpec((C, C), lambda b: (0, 0)),
            pl.BlockSpec((C,), lambda b: (0,)),
            pl.BlockSpec((C, 2 * C), lambda b: (0, 0)),
            pl.BlockSpec((2 * C,), lambda b: (0,)),
        ],
        out_specs=pl.BlockSpec((1, S, 2 * C), lambda b: (b, 0, 0)),
        out_shape=jax.ShapeDtypeStruct((B, S, 2 * C), F32),
    )(g, new_xyz, w0t, b0, w1t, b1, w2t, b2)


def kernel(xyz, points, W0, b0, W1, b1, W2, b2):
    xt = jnp.transpose(xyz, (2, 0, 1))
    nxt = _fps(xt)
    new_xyz = jnp.transpose(nxt, (1, 0, 2))
    w0t = W0.T
    w1t = W1.T
    w2t = W2.T
    g0 = _g0(xyz, points, w0t)
    idx = _select(xyz, jnp.transpose(nxt, (1, 2, 0)))
    g = _sc_gather(g0.reshape(B * N, C),
                   idx.reshape(NW, NCHUNK, CHUNK))
    out = _mlp(g.reshape(B, K, S, C), new_xyz,
               w0t, b0, w1t, b1, w2t, b2)
    return (new_xyz, out)
